# Initial kernel scaffold; baseline (speedup 1.0000x reference)
import jax
import jax.numpy as jnp
from jax.experimental import pallas as pl


def kernel(x, edge_index, batch, eps, W1, b1, gamma, beta, W2, b2, lin1_W, lin1_b, lin2_W, lin2_b):
    raise NotImplementedError("write your pallas kernel here")


# speedup vs baseline: 4.6202x; 4.6202x over previous
import functools

import jax
import jax.numpy as jnp
from jax import lax
from jax.experimental import pallas as pl
from jax.experimental.pallas import tpu as pltpu
import jax.experimental.pallas.tpu_sc as plsc

NC = 2
NS = 16
CH = 80


@functools.cache
def _agg_call(n_pad, d, e):
    nw = NC * NS
    assert e % nw == 0
    epw = e // nw
    nchunk = epw // CH
    rem = epw - nchunk * CH
    assert rem == 0, (e, epw, rem)
    assert n_pad % (NS * 8) == 0
    rows_per_tile = n_pad // NS

    mesh = plsc.VectorSubcoreMesh(core_axis_name="c", subcore_axis_name="s")

    @functools.partial(
        pl.kernel,
        out_type=jax.ShapeDtypeStruct((NC, n_pad, d), jnp.float32),
        mesh=mesh,
        scratch_types=[
            pltpu.VMEM((CH,), jnp.int32),
            pltpu.VMEM((CH,), jnp.int32),
            pltpu.VMEM((CH, d), jnp.float32),
            pltpu.VMEM_SHARED((n_pad, d), jnp.float32),
            pltpu.SemaphoreType.DMA,
        ],
    )
    def agg(h_hbm, src_hbm, dst_hbm, zeros_hbm, out_hbm, src_v, dst_v, rows_v,
            agg_sh, sem):
        c = lax.axis_index("c")
        s = lax.axis_index("s")
        wid = c * NS + s
        row0 = s * rows_per_tile
        pltpu.sync_copy(zeros_hbm.at[pl.ds(row0, rows_per_tile)],
                        agg_sh.at[pl.ds(row0, rows_per_tile)])
        plsc.subcore_barrier()
        ebase = wid * epw

        def body(i, carry):
            off = ebase + i * CH
            pltpu.sync_copy(src_hbm.at[pl.ds(off, CH)], src_v)
            pltpu.sync_copy(dst_hbm.at[pl.ds(off, CH)], dst_v)
            pltpu.async_copy(h_hbm.at[src_v], rows_v, sem).wait()
            pltpu.sync_copy(rows_v, agg_sh.at[dst_v], add=True)
            return carry

        lax.fori_loop(0, nchunk, body, 0)
        plsc.subcore_barrier()
        pltpu.sync_copy(agg_sh.at[pl.ds(row0, rows_per_tile)],
                        out_hbm.at[c, pl.ds(row0, rows_per_tile)])

    return agg


def _mlp_body(scale_ref, h_ref, agg_ref, w1_ref, b1_ref, g_ref, be_ref, w2_ref,
              b2_ref, out_ref):
    n = h_ref.shape[0]
    z = h_ref[:] * scale_ref[0, 0] + agg_ref[0, :n] + agg_ref[1, :n]
    z1 = jnp.dot(z, w1_ref[:], preferred_element_type=jnp.float32) + b1_ref[:]
    z1 = jnp.maximum(z1, 0.0)
    mu = jnp.mean(z1, axis=0, keepdims=True)
    cen = z1 - mu
    var = jnp.mean(cen * cen, axis=0, keepdims=True)
    z2 = cen * lax.rsqrt(var + 1e-5) * g_ref[:] + be_ref[:]
    z2 = jnp.maximum(z2, 0.0)
    z3 = jnp.dot(z2, w2_ref[:], preferred_element_type=jnp.float32) + b2_ref[:]
    out_ref[:] = jnp.maximum(z3, 0.0)


@functools.cache
def _mlp_call(n, d, h):
    return pl.pallas_call(
        _mlp_body,
        out_shape=jax.ShapeDtypeStruct((n, h), jnp.float32),
        in_specs=[
            pl.BlockSpec(memory_space=pltpu.SMEM),
            pl.BlockSpec(memory_space=pltpu.VMEM),
            pl.BlockSpec(memory_space=pltpu.VMEM),
            pl.BlockSpec(memory_space=pltpu.VMEM),
            pl.BlockSpec(memory_space=pltpu.VMEM),
            pl.BlockSpec(memory_space=pltpu.VMEM),
            pl.BlockSpec(memory_space=pltpu.VMEM),
            pl.BlockSpec(memory_space=pltpu.VMEM),
            pl.BlockSpec(memory_space=pltpu.VMEM),
        ],
        out_specs=pl.BlockSpec(memory_space=pltpu.VMEM),
    )


def _final_body(batch_ref, h_ref, w1_ref, b1_ref, w2_ref, b2_ref, out_ref, *,
                g):
    n = h_ref.shape[0]
    gids = lax.broadcasted_iota(jnp.int32, (g, n), 0)
    onehot = (batch_ref[:] == gids).astype(jnp.float32)
    gp = jnp.dot(onehot, h_ref[:], preferred_element_type=jnp.float32)
    g1 = jnp.dot(gp, w1_ref[:], preferred_element_type=jnp.float32) + b1_ref[:]
    g1 = jnp.maximum(g1, 0.0)
    out_ref[:] = (jnp.dot(g1, w2_ref[:], preferred_element_type=jnp.float32)
                  + b2_ref[:])


@functools.cache
def _final_call(n, h, o, g):
    return pl.pallas_call(
        functools.partial(_final_body, g=g),
        out_shape=jax.ShapeDtypeStruct((g, o), jnp.float32),
        in_specs=[pl.BlockSpec(memory_space=pltpu.VMEM)] * 6,
        out_specs=pl.BlockSpec(memory_space=pltpu.VMEM),
    )


def kernel(x, edge_index, batch, eps, W1, b1, gamma, beta, W2, b2, lin1_W,
           lin1_b, lin2_W, lin2_b):
    n, d = x.shape
    e = edge_index.shape[1]
    nlayers, _, hdim = W1.shape
    odim = lin2_W.shape[1]
    g = 64

    src = edge_index[0]
    dst = edge_index[1]
    n_pad = ((n + NS * 8 - 1) // (NS * 8)) * (NS * 8)
    zeros = jnp.zeros((n_pad, d), jnp.float32)
    agg_fn = _agg_call(n_pad, d, e)
    mlp_fn = _mlp_call(n, d, hdim)

    h = x
    for i in range(nlayers):
        agg = agg_fn(h, src, dst, zeros)
        scale = (1.0 + eps[i]).reshape(1, 1)
        h = mlp_fn(scale, h, agg, W1[i], b1[i].reshape(1, hdim),
                   gamma[i].reshape(1, hdim), beta[i].reshape(1, hdim), W2[i],
                   b2[i].reshape(1, hdim))

    return _final_call(n, hdim, odim, g)(
        batch.reshape(1, n), h, lin1_W, lin1_b.reshape(1, odim), lin2_W,
        lin2_b.reshape(1, odim))


# speedup vs baseline: 4.6925x; 1.0157x over previous
import functools

import jax
import jax.numpy as jnp
from jax import lax
from jax.experimental import pallas as pl
from jax.experimental.pallas import tpu as pltpu
import jax.experimental.pallas.tpu_sc as plsc

NC = 2
NS = 16
CH = 128


@functools.cache
def _agg_call(n_pad, d, epw_pad):
    assert epw_pad % CH == 0
    nchunk = epw_pad // CH
    assert n_pad % (NS * 8) == 0
    rows_per_tile = n_pad // NS

    mesh = plsc.VectorSubcoreMesh(core_axis_name="c", subcore_axis_name="s")

    @functools.partial(
        pl.kernel,
        out_type=jax.ShapeDtypeStruct((NC, n_pad, d), jnp.float32),
        mesh=mesh,
        scratch_types=[
            pltpu.VMEM((nchunk, CH), jnp.int32),
            pltpu.VMEM((nchunk, CH), jnp.int32),
            pltpu.VMEM((CH, d), jnp.float32),
            pltpu.VMEM_SHARED((n_pad, d), jnp.float32),
            pltpu.SemaphoreType.DMA,
        ],
    )
    def agg(h_hbm, src_hbm, dst_hbm, zeros_hbm, out_hbm, src_all, dst_all,
            buf, agg_sh, sem):
        c = lax.axis_index("c")
        s = lax.axis_index("s")
        wid = c * NS + s
        row0 = s * rows_per_tile
        pltpu.sync_copy(src_hbm.at[wid], src_all)
        pltpu.sync_copy(dst_hbm.at[wid], dst_all)
        pltpu.sync_copy(zeros_hbm.at[pl.ds(row0, rows_per_tile)],
                        agg_sh.at[pl.ds(row0, rows_per_tile)])
        plsc.subcore_barrier()

        def chunk(i, carry):
            pltpu.async_copy(h_hbm.at[src_all.at[i]], buf, sem).wait()
            pltpu.sync_copy(buf, agg_sh.at[dst_all.at[i]], add=True)
            return carry

        lax.fori_loop(0, nchunk, chunk, 0)
        plsc.subcore_barrier()
        pltpu.sync_copy(agg_sh.at[pl.ds(row0, rows_per_tile)],
                        out_hbm.at[c, pl.ds(row0, rows_per_tile)])

    return agg


def _mlp_body(scale_ref, h_ref, agg_ref, w1_ref, b1_ref, g_ref, be_ref, w2_ref,
              b2_ref, out_ref):
    n = h_ref.shape[0]
    z = h_ref[:] * scale_ref[0, 0] + agg_ref[0, :n] + agg_ref[1, :n]
    z1 = jnp.dot(z, w1_ref[:], preferred_element_type=jnp.float32) + b1_ref[:]
    z1 = jnp.maximum(z1, 0.0)
    mu = jnp.mean(z1, axis=0, keepdims=True)
    cen = z1 - mu
    var = jnp.mean(cen * cen, axis=0, keepdims=True)
    z2 = cen * lax.rsqrt(var + 1e-5) * g_ref[:] + be_ref[:]
    z2 = jnp.maximum(z2, 0.0)
    z3 = jnp.dot(z2, w2_ref[:], preferred_element_type=jnp.float32) + b2_ref[:]
    out_ref[:] = jnp.maximum(z3, 0.0)


@functools.cache
def _mlp_call(n, d, h):
    return pl.pallas_call(
        _mlp_body,
        out_shape=jax.ShapeDtypeStruct((n, h), jnp.float32),
        in_specs=[
            pl.BlockSpec(memory_space=pltpu.SMEM),
            pl.BlockSpec(memory_space=pltpu.VMEM),
            pl.BlockSpec(memory_space=pltpu.VMEM),
            pl.BlockSpec(memory_space=pltpu.VMEM),
            pl.BlockSpec(memory_space=pltpu.VMEM),
            pl.BlockSpec(memory_space=pltpu.VMEM),
            pl.BlockSpec(memory_space=pltpu.VMEM),
            pl.BlockSpec(memory_space=pltpu.VMEM),
            pl.BlockSpec(memory_space=pltpu.VMEM),
        ],
        out_specs=pl.BlockSpec(memory_space=pltpu.VMEM),
    )


def _final_body(batch_ref, h_ref, w1_ref, b1_ref, w2_ref, b2_ref, out_ref, *,
                g):
    n = h_ref.shape[0]
    gids = lax.broadcasted_iota(jnp.int32, (g, n), 0)
    onehot = (batch_ref[:] == gids).astype(jnp.float32)
    gp = jnp.dot(onehot, h_ref[:], preferred_element_type=jnp.float32)
    g1 = jnp.dot(gp, w1_ref[:], preferred_element_type=jnp.float32) + b1_ref[:]
    g1 = jnp.maximum(g1, 0.0)
    out_ref[:] = (jnp.dot(g1, w2_ref[:], preferred_element_type=jnp.float32)
                  + b2_ref[:])


@functools.cache
def _final_call(n, h, o, g):
    return pl.pallas_call(
        functools.partial(_final_body, g=g),
        out_shape=jax.ShapeDtypeStruct((g, o), jnp.float32),
        in_specs=[pl.BlockSpec(memory_space=pltpu.VMEM)] * 6,
        out_specs=pl.BlockSpec(memory_space=pltpu.VMEM),
    )


def kernel(x, edge_index, batch, eps, W1, b1, gamma, beta, W2, b2, lin1_W,
           lin1_b, lin2_W, lin2_b):
    n, d = x.shape
    e = edge_index.shape[1]
    nlayers, _, hdim = W1.shape
    odim = lin2_W.shape[1]
    g = 64

    nw = NC * NS
    epw = e // nw
    epw_pad = ((epw + CH - 1) // CH) * CH
    n_pad = ((n + NS * 8 - 1) // (NS * 8)) * (NS * 8)
    src = edge_index[0].reshape(nw, epw)
    dst = edge_index[1].reshape(nw, epw)
    pad = epw_pad - epw
    if pad:
        assert n_pad > n, "pad edges need an unused scatter row"
        src = jnp.pad(src, ((0, 0), (0, pad)))
        dst = jnp.pad(dst, ((0, 0), (0, pad)), constant_values=n_pad - 1)
    src = src.reshape(nw, epw_pad // CH, CH)
    dst = dst.reshape(nw, epw_pad // CH, CH)
    zeros = jnp.zeros((n_pad, d), jnp.float32)
    agg_fn = _agg_call(n_pad, d, epw_pad)
    mlp_fn = _mlp_call(n, d, hdim)

    h = x
    for i in range(nlayers):
        agg = agg_fn(h, src, dst, zeros)
        scale = (1.0 + eps[i]).reshape(1, 1)
        h = mlp_fn(scale, h, agg, W1[i], b1[i].reshape(1, hdim),
                   gamma[i].reshape(1, hdim), beta[i].reshape(1, hdim), W2[i],
                   b2[i].reshape(1, hdim))

    return _final_call(n, hdim, odim, g)(
        batch.reshape(1, n), h, lin1_W, lin1_b.reshape(1, odim), lin2_W,
        lin2_b.reshape(1, odim))
